# Initial kernel scaffold; baseline (speedup 1.0000x reference)
import jax
import jax.numpy as jnp
from jax.experimental import pallas as pl


def kernel(x, edge_index_r0, edge_index_r1, edge_index_r2, Wb0, coef0, Wb1, coef1, Wb2, coef2):
    raise NotImplementedError("write your pallas kernel here")


# speedup vs baseline: 3.8987x; 3.8987x over previous
import functools

import jax
import jax.numpy as jnp
from jax import lax
from jax.experimental import pallas as pl
from jax.experimental.pallas import tpu as pltpu
from jax.experimental.pallas import tpu_sc as plsc

N = 10000
D = 128
R = 3
E = 100000

NC = 2
NS = 16
NW = NC * NS
CH = 128
NCHUNK = 25
EW = NCHUNK * CH
E_PAD = EW * NW
N_PAD = 10240
RPS = N_PAD // NS
BLK = 640
GRID = N_PAD // BLK

def _sc_mesh():
    return plsc.VectorSubcoreMesh(core_axis_name="c", subcore_axis_name="s",
                                  num_cores=NC, num_subcores=NS)
_f32 = jnp.float32
_HIGH = lax.Precision.HIGHEST


def _pad_edges(idx):
    pad = E_PAD - E
    fill = (N + (jnp.arange(pad, dtype=jnp.int32) % 16)).astype(jnp.int32)
    return jnp.concatenate([idx.astype(jnp.int32), fill]).reshape(NW, NCHUNK, CH)


def _sc_hist(idx_all, ones6, z128):
    @functools.partial(
        pl.kernel,
        out_type=jax.ShapeDtypeStruct((NC, N_PAD, D), _f32),
        mesh=_sc_mesh(),
        scratch_types=[
            pltpu.VMEM((NCHUNK, CH), jnp.int32),
            pltpu.VMEM((CH, D), _f32),
            pltpu.VMEM_SHARED((N_PAD, D), _f32),
            pltpu.SemaphoreType.DMA,
        ],
    )
    def hist(idx_hbm, ones_hbm, z_hbm, out_hbm, idx_v, ones_v, table, sem):
        cid = lax.axis_index("c")
        sid = lax.axis_index("s")
        wid = sid * NC + cid
        row0 = sid * RPS

        @pl.loop(0, RPS // CH)
        def _zero(k):
            pltpu.sync_copy(z_hbm, table.at[pl.ds(row0 + k * CH, CH)])

        plsc.subcore_barrier()
        for h in range(6):
            pltpu.sync_copy(ones_hbm.at[h], ones_v)
            pltpu.sync_copy(idx_hbm.at[h, wid], idx_v)

            @pl.loop(0, NCHUNK)
            def _scat(j):
                pltpu.sync_copy(ones_v, table.at[idx_v.at[j]], add=True)

        plsc.subcore_barrier()
        pltpu.sync_copy(table.at[pl.ds(row0, RPS)],
                        out_hbm.at[cid, pl.ds(row0, RPS)])

    return hist(idx_all, ones6, z128)


def _sc_gather_scatter(hs0, hs1, hs2, idx_all, z128):
    @functools.partial(
        pl.kernel,
        out_type=[jax.ShapeDtypeStruct((NC, N_PAD, D), _f32)] * R,
        mesh=_sc_mesh(),
        scratch_types=[
            pltpu.VMEM((NCHUNK, CH), jnp.int32),
            pltpu.VMEM((NCHUNK, CH), jnp.int32),
            pltpu.VMEM((CH, D), _f32),
            pltpu.VMEM_SHARED((N_PAD, D), _f32),
            pltpu.SemaphoreType.DMA,
        ],
    )
    def gs(hs0_hbm, hs1_hbm, hs2_hbm, idx_hbm, z_hbm, p0, p1, p2,
           src_v, dst_v, rows, acc, sem):
        cid = lax.axis_index("c")
        sid = lax.axis_index("s")
        wid = sid * NC + cid
        row0 = sid * RPS
        for r, (hs, pout) in enumerate(((hs0_hbm, p0), (hs1_hbm, p1),
                                        (hs2_hbm, p2))):
            @pl.loop(0, RPS // CH)
            def _zero(k):
                pltpu.sync_copy(z_hbm, acc.at[pl.ds(row0 + k * CH, CH)])

            pltpu.sync_copy(idx_hbm.at[r, wid], src_v)
            pltpu.sync_copy(idx_hbm.at[R + r, wid], dst_v)
            plsc.subcore_barrier()

            @pl.loop(0, NCHUNK)
            def _edge(j):
                pltpu.async_copy(hs.at[src_v.at[j]], rows, sem).wait()
                pltpu.sync_copy(rows, acc.at[dst_v.at[j]], add=True)

            plsc.subcore_barrier()
            pltpu.sync_copy(acc.at[pl.ds(row0, RPS)],
                            pout.at[cid, pl.ds(row0, RPS)])
            plsc.subcore_barrier()

    return gs(hs0, hs1, hs2, idx_all, z128)


def _tc_basis(x_pad, Wb):
    def body(x_ref, wb_ref, out_ref):
        x = x_ref[...]
        for b in range(2):
            out_ref[b] = lax.dot_general(x, wb_ref[b], (((1,), (0,)), ((), ())),
                                         precision=_HIGH)

    return pl.pallas_call(
        body,
        grid=(GRID,),
        in_specs=[
            pl.BlockSpec((BLK, D), lambda i: (i, 0)),
            pl.BlockSpec((2, D, D), lambda i: (0, 0, 0)),
        ],
        out_specs=pl.BlockSpec((2, BLK, D), lambda i: (0, i, 0)),
        out_shape=jax.ShapeDtypeStruct((2, N_PAD, D), _f32),
    )(x_pad, Wb)


def _tc_degs(hist):
    def body(h_ref, out_ref):
        s = h_ref[0] + h_ref[1]
        cols = [lax.rsqrt(jnp.maximum(s[:, 16 * h:16 * h + 1], 1.0))
                for h in range(6)]
        cols.append(jnp.zeros((BLK, 2), _f32))
        out_ref[...] = jnp.concatenate(cols, axis=1)

    return pl.pallas_call(
        body,
        grid=(GRID,),
        in_specs=[pl.BlockSpec((NC, BLK, D), lambda i: (0, i, 0))],
        out_specs=pl.BlockSpec((BLK, 8), lambda i: (i, 0)),
        out_shape=jax.ShapeDtypeStruct((N_PAD, 8), _f32),
    )(hist)


def _tc_mk_hs(hb, coef, dinvt):
    def body(hb_ref, dinv_ref, coef_ref, o0, o1, o2):
        hb0 = hb_ref[0]
        hb1 = hb_ref[1]
        for r, o in enumerate((o0, o1, o2)):
            t = coef_ref[r, 0] * hb0 + coef_ref[r, 1] * hb1
            o[...] = dinv_ref[:, r:r + 1] * t

    return pl.pallas_call(
        body,
        grid=(GRID,),
        in_specs=[
            pl.BlockSpec((2, BLK, D), lambda i: (0, i, 0)),
            pl.BlockSpec((BLK, 8), lambda i: (i, 0)),
            pl.BlockSpec(memory_space=pltpu.SMEM),
        ],
        out_specs=[pl.BlockSpec((BLK, D), lambda i: (i, 0))] * R,
        out_shape=[jax.ShapeDtypeStruct((N_PAD, D), _f32)] * R,
    )(hb, dinvt, coef)


def _tc_combine(parts, dinvt, Wb, coef, last):
    def body(p0_ref, p1_ref, p2_ref, dinv_ref, wb_ref, coef_ref, *outs):
        h = jnp.zeros((BLK, D), _f32)
        for r, p in enumerate((p0_ref, p1_ref, p2_ref)):
            h = h + dinv_ref[:, R + r:R + r + 1] * (p[0] + p[1])
        if last:
            outs[0][...] = h
            return
        h = jnp.maximum(h, 0.0)
        hb = [lax.dot_general(h, wb_ref[b], (((1,), (0,)), ((), ())),
                              precision=_HIGH) for b in range(2)]
        for r, o in enumerate(outs):
            t = coef_ref[r, 0] * hb[0] + coef_ref[r, 1] * hb[1]
            o[...] = dinv_ref[:, r:r + 1] * t

    n_out = 1 if last else R
    return pl.pallas_call(
        body,
        grid=(GRID,),
        in_specs=[
            pl.BlockSpec((2, BLK, D), lambda i: (0, i, 0)),
            pl.BlockSpec((2, BLK, D), lambda i: (0, i, 0)),
            pl.BlockSpec((2, BLK, D), lambda i: (0, i, 0)),
            pl.BlockSpec((BLK, 8), lambda i: (i, 0)),
            pl.BlockSpec((2, D, D), lambda i: (0, 0, 0)),
            pl.BlockSpec(memory_space=pltpu.SMEM),
        ],
        out_specs=[pl.BlockSpec((BLK, D), lambda i: (i, 0))] * n_out,
        out_shape=[jax.ShapeDtypeStruct((N_PAD, D), _f32)] * n_out,
    )(parts[0], parts[1], parts[2], dinvt, Wb, coef)


def kernel(x, edge_index_r0, edge_index_r1, edge_index_r2,
           Wb0, coef0, Wb1, coef1, Wb2, coef2):
    idx_all = jnp.stack([
        _pad_edges(edge_index_r0[0]),
        _pad_edges(edge_index_r1[0]),
        _pad_edges(edge_index_r2[0]),
        _pad_edges(edge_index_r0[1]),
        _pad_edges(edge_index_r1[1]),
        _pad_edges(edge_index_r2[1]),
    ])
    lane_grp = jnp.arange(D, dtype=jnp.int32) // 16
    ones6 = (lane_grp[None, :] == jnp.arange(6, dtype=jnp.int32)[:, None])
    ones6 = jnp.broadcast_to(ones6[:, None, :].astype(_f32), (6, CH, D))
    z128 = jnp.zeros((CH, D), _f32)
    x_pad = jnp.pad(x, ((0, N_PAD - N), (0, 0)))

    hist = _sc_hist(idx_all, ones6, z128)
    hb = _tc_basis(x_pad, Wb0)
    dinvt = _tc_degs(hist)

    hs = _tc_mk_hs(hb, coef0, dinvt)
    parts = _sc_gather_scatter(hs[0], hs[1], hs[2], idx_all, z128)
    hs = _tc_combine(parts, dinvt, Wb1, coef1, last=False)
    parts = _sc_gather_scatter(hs[0], hs[1], hs[2], idx_all, z128)
    hs = _tc_combine(parts, dinvt, Wb2, coef2, last=False)
    parts = _sc_gather_scatter(hs[0], hs[1], hs[2], idx_all, z128)
    out = _tc_combine(parts, dinvt, jnp.zeros((2, D, D), _f32),
                      jnp.zeros((3, 2), _f32), last=True)
    return out[0][:N]


# speedup vs baseline: 4.8503x; 1.2441x over previous
import functools

import jax
import jax.numpy as jnp
from jax import lax
from jax.experimental import pallas as pl
from jax.experimental.pallas import tpu as pltpu
from jax.experimental.pallas import tpu_sc as plsc

N = 10000
D = 128
R = 3
E = 100000

NC = 2
NS = 16
NW = NC * NS
CH = 128
NCHUNK = 25
EW = NCHUNK * CH
E_PAD = EW * NW
N_PAD = 10240
RPS = N_PAD // NS
BLK = 640
GRID = N_PAD // BLK

def _sc_mesh():
    return plsc.VectorSubcoreMesh(core_axis_name="c", subcore_axis_name="s",
                                  num_cores=NC, num_subcores=NS)
_f32 = jnp.float32
_HIGH = lax.Precision.HIGHEST


def _pad_edges(idx):
    pad = E_PAD - E
    fill = (N + (jnp.arange(pad, dtype=jnp.int32) % 16)).astype(jnp.int32)
    return jnp.concatenate([idx.astype(jnp.int32), fill]).reshape(NW, NCHUNK, CH)


def _sc_hist(idx_all, ones6, z128):
    @functools.partial(
        pl.kernel,
        out_type=jax.ShapeDtypeStruct((NC, N_PAD, D), _f32),
        mesh=_sc_mesh(),
        scratch_types=[
            pltpu.VMEM((NCHUNK, CH), jnp.int32),
            pltpu.VMEM((CH, D), _f32),
            pltpu.VMEM_SHARED((N_PAD, D), _f32),
            pltpu.SemaphoreType.DMA,
        ],
    )
    def hist(idx_hbm, ones_hbm, z_hbm, out_hbm, idx_v, ones_v, table, sem):
        cid = lax.axis_index("c")
        sid = lax.axis_index("s")
        wid = sid * NC + cid
        row0 = sid * RPS

        @pl.loop(0, RPS // CH)
        def _zero(k):
            pltpu.sync_copy(z_hbm, table.at[pl.ds(row0 + k * CH, CH)])

        plsc.subcore_barrier()
        for h in range(6):
            pltpu.sync_copy(ones_hbm.at[h], ones_v)
            pltpu.sync_copy(idx_hbm.at[h, wid], idx_v)

            @pl.loop(0, NCHUNK)
            def _scat(j):
                pltpu.sync_copy(ones_v, table.at[idx_v.at[j]], add=True)

        plsc.subcore_barrier()
        pltpu.sync_copy(table.at[pl.ds(row0, RPS)],
                        out_hbm.at[cid, pl.ds(row0, RPS)])

    return hist(idx_all, ones6, z128)


def _sc_gather_scatter(hs0, hs1, hs2, idx_all, z128):
    @functools.partial(
        pl.kernel,
        out_type=[jax.ShapeDtypeStruct((NC, N_PAD, D), _f32)] * R,
        mesh=_sc_mesh(),
        scratch_types=[
            pltpu.VMEM((NCHUNK, CH), jnp.int32),
            pltpu.VMEM((NCHUNK, CH), jnp.int32),
            pltpu.VMEM((CH, D), _f32),
            pltpu.VMEM((CH, D), _f32),
            pltpu.VMEM_SHARED((N_PAD, D), _f32),
            pltpu.SemaphoreType.DMA,
            pltpu.SemaphoreType.DMA,
        ],
    )
    def gs(hs0_hbm, hs1_hbm, hs2_hbm, idx_hbm, z_hbm, p0, p1, p2,
           src_v, dst_v, rows0, rows1, acc, sem0, sem1):
        cid = lax.axis_index("c")
        sid = lax.axis_index("s")
        wid = sid * NC + cid
        row0 = sid * RPS
        rows = (rows0, rows1)
        sems = (sem0, sem1)
        for r, (hs, pout) in enumerate(((hs0_hbm, p0), (hs1_hbm, p1),
                                        (hs2_hbm, p2))):
            @pl.loop(0, RPS // CH)
            def _zero(k):
                pltpu.sync_copy(z_hbm, acc.at[pl.ds(row0 + k * CH, CH)])

            pltpu.sync_copy(idx_hbm.at[r, wid], src_v)
            pltpu.sync_copy(idx_hbm.at[R + r, wid], dst_v)
            plsc.subcore_barrier()

            cps = [pltpu.async_copy(hs.at[src_v.at[0]], rows[0], sems[0]),
                   None]
            for j in range(NCHUNK):
                b = j % 2
                if j + 1 < NCHUNK:
                    cps[1 - b] = pltpu.async_copy(hs.at[src_v.at[j + 1]],
                                                  rows[1 - b], sems[1 - b])
                cps[b].wait()
                pltpu.sync_copy(rows[b], acc.at[dst_v.at[j]], add=True)

            plsc.subcore_barrier()
            pltpu.sync_copy(acc.at[pl.ds(row0, RPS)],
                            pout.at[cid, pl.ds(row0, RPS)])
            plsc.subcore_barrier()

    return gs(hs0, hs1, hs2, idx_all, z128)


def _tc_basis(x_pad, Wb):
    def body(x_ref, wb_ref, out_ref):
        x = x_ref[...]
        for b in range(2):
            out_ref[b] = lax.dot_general(x, wb_ref[b], (((1,), (0,)), ((), ())),
                                         precision=_HIGH)

    return pl.pallas_call(
        body,
        grid=(GRID,),
        in_specs=[
            pl.BlockSpec((BLK, D), lambda i: (i, 0)),
            pl.BlockSpec((2, D, D), lambda i: (0, 0, 0)),
        ],
        out_specs=pl.BlockSpec((2, BLK, D), lambda i: (0, i, 0)),
        out_shape=jax.ShapeDtypeStruct((2, N_PAD, D), _f32),
    )(x_pad, Wb)


def _tc_degs(hist):
    def body(h_ref, out_ref):
        s = h_ref[0] + h_ref[1]
        cols = [lax.rsqrt(jnp.maximum(s[:, 16 * h:16 * h + 1], 1.0))
                for h in range(6)]
        cols.append(jnp.zeros((BLK, 2), _f32))
        out_ref[...] = jnp.concatenate(cols, axis=1)

    return pl.pallas_call(
        body,
        grid=(GRID,),
        in_specs=[pl.BlockSpec((NC, BLK, D), lambda i: (0, i, 0))],
        out_specs=pl.BlockSpec((BLK, 8), lambda i: (i, 0)),
        out_shape=jax.ShapeDtypeStruct((N_PAD, 8), _f32),
    )(hist)


def _tc_mk_hs(hb, coef, dinvt):
    def body(hb_ref, dinv_ref, coef_ref, o0, o1, o2):
        hb0 = hb_ref[0]
        hb1 = hb_ref[1]
        for r, o in enumerate((o0, o1, o2)):
            t = coef_ref[r, 0] * hb0 + coef_ref[r, 1] * hb1
            o[...] = dinv_ref[:, r:r + 1] * t

    return pl.pallas_call(
        body,
        grid=(GRID,),
        in_specs=[
            pl.BlockSpec((2, BLK, D), lambda i: (0, i, 0)),
            pl.BlockSpec((BLK, 8), lambda i: (i, 0)),
            pl.BlockSpec(memory_space=pltpu.SMEM),
        ],
        out_specs=[pl.BlockSpec((BLK, D), lambda i: (i, 0))] * R,
        out_shape=[jax.ShapeDtypeStruct((N_PAD, D), _f32)] * R,
    )(hb, dinvt, coef)


def _tc_combine(parts, dinvt, Wb, coef, last):
    def body(p0_ref, p1_ref, p2_ref, dinv_ref, wb_ref, coef_ref, *outs):
        h = jnp.zeros((BLK, D), _f32)
        for r, p in enumerate((p0_ref, p1_ref, p2_ref)):
            h = h + dinv_ref[:, R + r:R + r + 1] * (p[0] + p[1])
        if last:
            outs[0][...] = h
            return
        h = jnp.maximum(h, 0.0)
        hb = [lax.dot_general(h, wb_ref[b], (((1,), (0,)), ((), ())),
                              precision=_HIGH) for b in range(2)]
        for r, o in enumerate(outs):
            t = coef_ref[r, 0] * hb[0] + coef_ref[r, 1] * hb[1]
            o[...] = dinv_ref[:, r:r + 1] * t

    n_out = 1 if last else R
    return pl.pallas_call(
        body,
        grid=(GRID,),
        in_specs=[
            pl.BlockSpec((2, BLK, D), lambda i: (0, i, 0)),
            pl.BlockSpec((2, BLK, D), lambda i: (0, i, 0)),
            pl.BlockSpec((2, BLK, D), lambda i: (0, i, 0)),
            pl.BlockSpec((BLK, 8), lambda i: (i, 0)),
            pl.BlockSpec((2, D, D), lambda i: (0, 0, 0)),
            pl.BlockSpec(memory_space=pltpu.SMEM),
        ],
        out_specs=[pl.BlockSpec((BLK, D), lambda i: (i, 0))] * n_out,
        out_shape=[jax.ShapeDtypeStruct((N_PAD, D), _f32)] * n_out,
    )(parts[0], parts[1], parts[2], dinvt, Wb, coef)


def kernel(x, edge_index_r0, edge_index_r1, edge_index_r2,
           Wb0, coef0, Wb1, coef1, Wb2, coef2):
    idx_all = jnp.stack([
        _pad_edges(edge_index_r0[0]),
        _pad_edges(edge_index_r1[0]),
        _pad_edges(edge_index_r2[0]),
        _pad_edges(edge_index_r0[1]),
        _pad_edges(edge_index_r1[1]),
        _pad_edges(edge_index_r2[1]),
    ])
    lane_grp = jnp.arange(D, dtype=jnp.int32) // 16
    ones6 = (lane_grp[None, :] == jnp.arange(6, dtype=jnp.int32)[:, None])
    ones6 = jnp.broadcast_to(ones6[:, None, :].astype(_f32), (6, CH, D))
    z128 = jnp.zeros((CH, D), _f32)
    x_pad = jnp.pad(x, ((0, N_PAD - N), (0, 0)))

    hist = _sc_hist(idx_all, ones6, z128)
    hb = _tc_basis(x_pad, Wb0)
    dinvt = _tc_degs(hist)

    hs = _tc_mk_hs(hb, coef0, dinvt)
    parts = _sc_gather_scatter(hs[0], hs[1], hs[2], idx_all, z128)
    hs = _tc_combine(parts, dinvt, Wb1, coef1, last=False)
    parts = _sc_gather_scatter(hs[0], hs[1], hs[2], idx_all, z128)
    hs = _tc_combine(parts, dinvt, Wb2, coef2, last=False)
    parts = _sc_gather_scatter(hs[0], hs[1], hs[2], idx_all, z128)
    out = _tc_combine(parts, dinvt, jnp.zeros((2, D, D), _f32),
                      jnp.zeros((3, 2), _f32), last=True)
    return out[0][:N]


# speedup vs baseline: 5.2696x; 1.0865x over previous
import functools

import jax
import jax.numpy as jnp
from jax import lax
from jax.experimental import pallas as pl
from jax.experimental.pallas import tpu as pltpu
from jax.experimental.pallas import tpu_sc as plsc

N = 10000
D = 128
R = 3
E = 100000

NC = 2
NS = 16
NW = NC * NS
CH = 128
NCHUNK = 25
EW = NCHUNK * CH
E_PAD = EW * NW
N_PAD = 10240
RPS = N_PAD // NS
BLK = 640
GRID = N_PAD // BLK

def _sc_mesh():
    return plsc.VectorSubcoreMesh(core_axis_name="c", subcore_axis_name="s",
                                  num_cores=NC, num_subcores=NS)
_f32 = jnp.float32
_HIGH = lax.Precision.HIGHEST


def _pad_edges(idx):
    pad = E_PAD - E
    fill = (N + (jnp.arange(pad, dtype=jnp.int32) % 16)).astype(jnp.int32)
    return jnp.concatenate([idx.astype(jnp.int32), fill]).reshape(NW, NCHUNK, CH)


NVEC = EW // 16


def _sc_hist(idx16_all):
    HR = 6 * N_PAD // 128

    @functools.partial(
        pl.kernel,
        out_type=jax.ShapeDtypeStruct((NW, HR, 128), _f32),
        mesh=_sc_mesh(),
        compiler_params=pltpu.CompilerParams(needs_layout_passes=False),
        scratch_types=[
            pltpu.VMEM((6 * NCHUNK, CH), jnp.int32),
            pltpu.VMEM((HR, 128), _f32),
            pltpu.SemaphoreType.DMA,
        ],
    )
    def hist(idx_hbm, out_hbm, idx_v, hcnt, sem):
        cid = lax.axis_index("c")
        sid = lax.axis_index("s")
        wid = sid * NC + cid
        for h in range(6):
            pltpu.sync_copy(idx_hbm.at[h, wid],
                            idx_v.at[pl.ds(h * NCHUNK, NCHUNK)])

        zeros = jnp.zeros((16,), _f32)

        @pl.loop(0, HR * 8)
        def _z(t):
            hcnt[t // 8, pl.ds((t % 8) * 16, 16)] = zeros

        ones = jnp.ones((16,), _f32)

        @pl.loop(0, 6 * NCHUNK)
        def _acc(r):
            base = (r // NCHUNK) * N_PAD
            for k in range(8):
                v = idx_v[r, pl.ds(k * 16, 16)] + base
                plsc.addupdate_scatter(
                    hcnt,
                    [lax.shift_right_logical(v, 7), v & 127],
                    ones)

        pltpu.sync_copy(hcnt, out_hbm.at[wid])

    return hist(idx16_all)


def _sc_gather_scatter(hs0, hs1, hs2, idx_all, z128):
    @functools.partial(
        pl.kernel,
        out_type=[jax.ShapeDtypeStruct((NC, N_PAD, D), _f32)] * R,
        mesh=_sc_mesh(),
        scratch_types=[
            pltpu.VMEM((NCHUNK, CH), jnp.int32),
            pltpu.VMEM((NCHUNK, CH), jnp.int32),
            pltpu.VMEM((CH, D), _f32),
            pltpu.VMEM((CH, D), _f32),
            pltpu.VMEM_SHARED((N_PAD, D), _f32),
            pltpu.SemaphoreType.DMA,
            pltpu.SemaphoreType.DMA,
        ],
    )
    def gs(hs0_hbm, hs1_hbm, hs2_hbm, idx_hbm, z_hbm, p0, p1, p2,
           src_v, dst_v, rows0, rows1, acc, *gsems):
        rows = (rows0, rows1)
        cid = lax.axis_index("c")
        sid = lax.axis_index("s")
        wid = sid * NC + cid
        row0 = sid * RPS
        NB, LA = 2, 1
        for r, (hs, pout) in enumerate(((hs0_hbm, p0), (hs1_hbm, p1),
                                        (hs2_hbm, p2))):
            @pl.loop(0, RPS // CH)
            def _zero(k):
                pltpu.sync_copy(z_hbm, acc.at[pl.ds(row0 + k * CH, CH)])

            pltpu.sync_copy(idx_hbm.at[r, wid], src_v)
            pltpu.sync_copy(idx_hbm.at[R + r, wid], dst_v)
            plsc.subcore_barrier()

            g = [None] * NCHUNK
            for t in range(min(LA, NCHUNK)):
                g[t] = pltpu.async_copy(hs.at[src_v.at[t]],
                                        rows[t % NB], gsems[t % NB])
            for j in range(NCHUNK):
                if j + LA < NCHUNK:
                    b = (j + LA) % NB
                    g[j + LA] = pltpu.async_copy(hs.at[src_v.at[j + LA]],
                                                 rows[b], gsems[b])
                g[j].wait()
                pltpu.sync_copy(rows[j % NB], acc.at[dst_v.at[j]],
                                add=True)

            plsc.subcore_barrier()
            pltpu.sync_copy(acc.at[pl.ds(row0, RPS)],
                            pout.at[cid, pl.ds(row0, RPS)])
            plsc.subcore_barrier()

    return gs(hs0, hs1, hs2, idx_all, z128)


def _tc_basis(x_pad, Wb):
    def body(x_ref, wb_ref, out_ref):
        x = x_ref[...]
        for b in range(2):
            out_ref[b] = lax.dot_general(x, wb_ref[b], (((1,), (0,)), ((), ())),
                                         precision=_HIGH)

    return pl.pallas_call(
        body,
        grid=(GRID,),
        in_specs=[
            pl.BlockSpec((BLK, D), lambda i: (i, 0)),
            pl.BlockSpec((2, D, D), lambda i: (0, 0, 0)),
        ],
        out_specs=pl.BlockSpec((2, BLK, D), lambda i: (0, i, 0)),
        out_shape=jax.ShapeDtypeStruct((2, N_PAD, D), _f32),
    )(x_pad, Wb)


def _tc_degs(hist2, sel):
    def body(t_ref, s_ref, out_ref):
        d = lax.dot_general(t_ref[...], s_ref[...],
                            (((0,), (0,)), ((), ())), precision=_HIGH)
        out_ref[...] = lax.rsqrt(jnp.maximum(d, 1.0))

    return pl.pallas_call(
        body,
        grid=(GRID,),
        in_specs=[
            pl.BlockSpec((192, BLK), lambda i: (0, i)),
            pl.BlockSpec((192, 8), lambda i: (0, 0)),
        ],
        out_specs=pl.BlockSpec((BLK, 8), lambda i: (i, 0)),
        out_shape=jax.ShapeDtypeStruct((N_PAD, 8), _f32),
    )(hist2, sel)


def _tc_mk_hs(hb, coef, dinvt):
    def body(hb_ref, dinv_ref, coef_ref, o0, o1, o2):
        hb0 = hb_ref[0]
        hb1 = hb_ref[1]
        for r, o in enumerate((o0, o1, o2)):
            t = coef_ref[r, 0] * hb0 + coef_ref[r, 1] * hb1
            o[...] = dinv_ref[:, r:r + 1] * t

    return pl.pallas_call(
        body,
        grid=(GRID,),
        in_specs=[
            pl.BlockSpec((2, BLK, D), lambda i: (0, i, 0)),
            pl.BlockSpec((BLK, 8), lambda i: (i, 0)),
            pl.BlockSpec(memory_space=pltpu.SMEM),
        ],
        out_specs=[pl.BlockSpec((BLK, D), lambda i: (i, 0))] * R,
        out_shape=[jax.ShapeDtypeStruct((N_PAD, D), _f32)] * R,
    )(hb, dinvt, coef)


def _tc_combine(parts, dinvt, Wb, coef, last):
    def body(p0_ref, p1_ref, p2_ref, dinv_ref, wb_ref, coef_ref, *outs):
        h = jnp.zeros((BLK, D), _f32)
        for r, p in enumerate((p0_ref, p1_ref, p2_ref)):
            h = h + dinv_ref[:, R + r:R + r + 1] * (p[0] + p[1])
        if last:
            outs[0][...] = h
            return
        h = jnp.maximum(h, 0.0)
        hb = [lax.dot_general(h, wb_ref[b], (((1,), (0,)), ((), ())),
                              precision=_HIGH) for b in range(2)]
        for r, o in enumerate(outs):
            t = coef_ref[r, 0] * hb[0] + coef_ref[r, 1] * hb[1]
            o[...] = dinv_ref[:, r:r + 1] * t

    n_out = 1 if last else R
    return pl.pallas_call(
        body,
        grid=(GRID,),
        in_specs=[
            pl.BlockSpec((2, BLK, D), lambda i: (0, i, 0)),
            pl.BlockSpec((2, BLK, D), lambda i: (0, i, 0)),
            pl.BlockSpec((2, BLK, D), lambda i: (0, i, 0)),
            pl.BlockSpec((BLK, 8), lambda i: (i, 0)),
            pl.BlockSpec((2, D, D), lambda i: (0, 0, 0)),
            pl.BlockSpec(memory_space=pltpu.SMEM),
        ],
        out_specs=[pl.BlockSpec((BLK, D), lambda i: (i, 0))] * n_out,
        out_shape=[jax.ShapeDtypeStruct((N_PAD, D), _f32)] * n_out,
    )(parts[0], parts[1], parts[2], dinvt, Wb, coef)


def kernel(x, edge_index_r0, edge_index_r1, edge_index_r2,
           Wb0, coef0, Wb1, coef1, Wb2, coef2):
    idx_all = jnp.stack([
        _pad_edges(edge_index_r0[0]),
        _pad_edges(edge_index_r1[0]),
        _pad_edges(edge_index_r2[0]),
        _pad_edges(edge_index_r0[1]),
        _pad_edges(edge_index_r1[1]),
        _pad_edges(edge_index_r2[1]),
    ])
    hvec = jnp.tile(jnp.arange(6, dtype=jnp.int32), NW)
    sel = (hvec[:, None] == jnp.arange(8, dtype=jnp.int32)[None, :])
    sel = sel.astype(_f32)
    z128 = jnp.zeros((CH, D), _f32)
    x_pad = jnp.pad(x, ((0, N_PAD - N), (0, 0)))

    hist = _sc_hist(idx_all)
    hb = _tc_basis(x_pad, Wb0)
    dinvt = _tc_degs(hist.reshape(NC * NS * 6, N_PAD), sel)

    hs = _tc_mk_hs(hb, coef0, dinvt)
    parts = _sc_gather_scatter(hs[0], hs[1], hs[2], idx_all, z128)
    hs = _tc_combine(parts, dinvt, Wb1, coef1, last=False)
    parts = _sc_gather_scatter(hs[0], hs[1], hs[2], idx_all, z128)
    hs = _tc_combine(parts, dinvt, Wb2, coef2, last=False)
    parts = _sc_gather_scatter(hs[0], hs[1], hs[2], idx_all, z128)
    out = _tc_combine(parts, dinvt, jnp.zeros((2, D, D), _f32),
                      jnp.zeros((3, 2), _f32), last=True)
    return out[0][:N]


# speedup vs baseline: 5.4647x; 1.0370x over previous
import functools

import jax
import jax.numpy as jnp
from jax import lax
from jax.experimental import pallas as pl
from jax.experimental.pallas import tpu as pltpu
from jax.experimental.pallas import tpu_sc as plsc

N = 10000
D = 128
R = 3
E = 100000

NC = 2
NS = 16
NW = NC * NS
CH = 128
NCHUNK = 25
EW = NCHUNK * CH
E_PAD = EW * NW
N_PAD = 10240
RPS = N_PAD // NS
BLK = 640
GRID = N_PAD // BLK

def _sc_mesh():
    return plsc.VectorSubcoreMesh(core_axis_name="c", subcore_axis_name="s",
                                  num_cores=NC, num_subcores=NS)
_f32 = jnp.float32
_HIGH = lax.Precision.HIGHEST


def _pad_edges(idx):
    pad = E_PAD - E
    fill = (N + (jnp.arange(pad, dtype=jnp.int32) % 16)).astype(jnp.int32)
    return jnp.concatenate([idx.astype(jnp.int32), fill]).reshape(NW, NCHUNK, CH)


NVEC = EW // 16


def _sc_hist(idx16_all):
    HR = 6 * N_PAD // 128

    @functools.partial(
        pl.kernel,
        out_type=jax.ShapeDtypeStruct((NW, HR, 128), _f32),
        mesh=_sc_mesh(),
        compiler_params=pltpu.CompilerParams(needs_layout_passes=False),
        scratch_types=[
            pltpu.VMEM((6 * NCHUNK, CH), jnp.int32),
            pltpu.VMEM((HR, 128), _f32),
            pltpu.SemaphoreType.DMA,
        ],
    )
    def hist(idx_hbm, out_hbm, idx_v, hcnt, sem):
        cid = lax.axis_index("c")
        sid = lax.axis_index("s")
        wid = sid * NC + cid
        for h in range(6):
            pltpu.sync_copy(idx_hbm.at[h, wid],
                            idx_v.at[pl.ds(h * NCHUNK, NCHUNK)])

        zeros = jnp.zeros((16,), _f32)

        @pl.loop(0, HR * 8)
        def _z(t):
            hcnt[t // 8, pl.ds((t % 8) * 16, 16)] = zeros

        ones = jnp.ones((16,), _f32)

        @pl.loop(0, 6 * NCHUNK)
        def _acc(r):
            base = (r // NCHUNK) * N_PAD
            for k in range(8):
                v = idx_v[r, pl.ds(k * 16, 16)] + base
                plsc.addupdate_scatter(
                    hcnt,
                    [lax.shift_right_logical(v, 7), v & 127],
                    ones)

        pltpu.sync_copy(hcnt, out_hbm.at[wid])

    return hist(idx16_all)


def _sc_gather_scatter(hs0, hs1, hs2, idx_all, z128):
    @functools.partial(
        pl.kernel,
        out_type=[jax.ShapeDtypeStruct((NC, N_PAD, D), _f32)] * R,
        mesh=_sc_mesh(),
        scratch_types=[
            pltpu.VMEM((NCHUNK, CH), jnp.int32),
            pltpu.VMEM((NCHUNK, CH), jnp.int32),
            pltpu.VMEM((CH, D), _f32),
            pltpu.VMEM((CH, D), _f32),
            pltpu.VMEM_SHARED((N_PAD, D), _f32),
            pltpu.SemaphoreType.DMA,
            pltpu.SemaphoreType.DMA,
        ],
    )
    def gs(hs0_hbm, hs1_hbm, hs2_hbm, idx_hbm, z_hbm, p0, p1, p2,
           src_v, dst_v, rows0, rows1, acc, *gsems):
        rows = (rows0, rows1)
        cid = lax.axis_index("c")
        sid = lax.axis_index("s")
        wid = sid * NC + cid
        row0 = sid * RPS
        NB, LA = 2, 1
        for r, (hs, pout) in enumerate(((hs0_hbm, p0), (hs1_hbm, p1),
                                        (hs2_hbm, p2))):
            @pl.loop(0, RPS // CH)
            def _zero(k):
                pltpu.sync_copy(z_hbm, acc.at[pl.ds(row0 + k * CH, CH)])

            pltpu.sync_copy(idx_hbm.at[r, wid], src_v)
            pltpu.sync_copy(idx_hbm.at[R + r, wid], dst_v)
            plsc.subcore_barrier()

            g = [None] * NCHUNK
            for t in range(min(LA, NCHUNK)):
                g[t] = pltpu.async_copy(hs.at[src_v.at[t]],
                                        rows[t % NB], gsems[t % NB])
            for j in range(NCHUNK):
                if j + LA < NCHUNK:
                    b = (j + LA) % NB
                    g[j + LA] = pltpu.async_copy(hs.at[src_v.at[j + LA]],
                                                 rows[b], gsems[b])
                g[j].wait()
                pltpu.sync_copy(rows[j % NB], acc.at[dst_v.at[j]],
                                add=True)

            plsc.subcore_barrier()
            pltpu.sync_copy(acc.at[pl.ds(row0, RPS)],
                            pout.at[cid, pl.ds(row0, RPS)])
            plsc.subcore_barrier()

    return gs(hs0, hs1, hs2, idx_all, z128)


def _tc_basis(x_pad, W):
    def body(x_ref, w_ref, out_ref):
        x = x_ref[...]
        for r in range(R):
            out_ref[r] = lax.dot_general(x, w_ref[r], (((1,), (0,)), ((), ())),
                                         precision=_HIGH)

    return pl.pallas_call(
        body,
        grid=(GRID,),
        in_specs=[
            pl.BlockSpec((BLK, D), lambda i: (i, 0)),
            pl.BlockSpec((R, D, D), lambda i: (0, 0, 0)),
        ],
        out_specs=pl.BlockSpec((R, BLK, D), lambda i: (0, i, 0)),
        out_shape=jax.ShapeDtypeStruct((R, N_PAD, D), _f32),
    )(x_pad, W)


def _tc_degs(hist2, sel):
    def body(t_ref, s_ref, out_ref):
        d = lax.dot_general(t_ref[...], s_ref[...],
                            (((0,), (0,)), ((), ())), precision=_HIGH)
        out_ref[...] = lax.rsqrt(jnp.maximum(d, 1.0))

    return pl.pallas_call(
        body,
        grid=(GRID,),
        in_specs=[
            pl.BlockSpec((192, BLK), lambda i: (0, i)),
            pl.BlockSpec((192, 8), lambda i: (0, 0)),
        ],
        out_specs=pl.BlockSpec((BLK, 8), lambda i: (i, 0)),
        out_shape=jax.ShapeDtypeStruct((N_PAD, 8), _f32),
    )(hist2, sel)


def _tc_mk_hs(hb, dinvt):
    def body(hb_ref, dinv_ref, o0, o1, o2):
        for r, o in enumerate((o0, o1, o2)):
            o[...] = dinv_ref[:, r:r + 1] * hb_ref[r]

    return pl.pallas_call(
        body,
        grid=(GRID,),
        in_specs=[
            pl.BlockSpec((R, BLK, D), lambda i: (0, i, 0)),
            pl.BlockSpec((BLK, 8), lambda i: (i, 0)),
        ],
        out_specs=[pl.BlockSpec((BLK, D), lambda i: (i, 0))] * R,
        out_shape=[jax.ShapeDtypeStruct((N_PAD, D), _f32)] * R,
    )(hb, dinvt)


def _tc_combine(parts, dinvt, W, last):
    def body(p0_ref, p1_ref, p2_ref, dinv_ref, w_ref, *outs):
        h = jnp.zeros((BLK, D), _f32)
        for r, p in enumerate((p0_ref, p1_ref, p2_ref)):
            h = h + dinv_ref[:, R + r:R + r + 1] * (p[0] + p[1])
        if last:
            outs[0][...] = h
            return
        h = jnp.maximum(h, 0.0)
        for r, o in enumerate(outs):
            t = lax.dot_general(h, w_ref[r], (((1,), (0,)), ((), ())),
                                precision=_HIGH)
            o[...] = dinv_ref[:, r:r + 1] * t

    n_out = 1 if last else R
    return pl.pallas_call(
        body,
        grid=(GRID,),
        in_specs=[
            pl.BlockSpec((2, BLK, D), lambda i: (0, i, 0)),
            pl.BlockSpec((2, BLK, D), lambda i: (0, i, 0)),
            pl.BlockSpec((2, BLK, D), lambda i: (0, i, 0)),
            pl.BlockSpec((BLK, 8), lambda i: (i, 0)),
            pl.BlockSpec((R, D, D), lambda i: (0, 0, 0)),
        ],
        out_specs=[pl.BlockSpec((BLK, D), lambda i: (i, 0))] * n_out,
        out_shape=[jax.ShapeDtypeStruct((N_PAD, D), _f32)] * n_out,
    )(parts[0], parts[1], parts[2], dinvt, W)


def kernel(x, edge_index_r0, edge_index_r1, edge_index_r2,
           Wb0, coef0, Wb1, coef1, Wb2, coef2):
    idx_all = jnp.stack([
        _pad_edges(edge_index_r0[0]),
        _pad_edges(edge_index_r1[0]),
        _pad_edges(edge_index_r2[0]),
        _pad_edges(edge_index_r0[1]),
        _pad_edges(edge_index_r1[1]),
        _pad_edges(edge_index_r2[1]),
    ])
    hvec = jnp.tile(jnp.arange(6, dtype=jnp.int32), NW)
    sel = (hvec[:, None] == jnp.arange(8, dtype=jnp.int32)[None, :])
    sel = sel.astype(_f32)
    z128 = jnp.zeros((CH, D), _f32)
    x_pad = jnp.pad(x, ((0, N_PAD - N), (0, 0)))

    W0 = jnp.einsum('rb,bio->rio', coef0, Wb0)
    W1 = jnp.einsum('rb,bio->rio', coef1, Wb1)
    W2 = jnp.einsum('rb,bio->rio', coef2, Wb2)

    hist = _sc_hist(idx_all)
    hb = _tc_basis(x_pad, W0)
    dinvt = _tc_degs(hist.reshape(NC * NS * 6, N_PAD), sel)

    hs = _tc_mk_hs(hb, dinvt)
    parts = _sc_gather_scatter(hs[0], hs[1], hs[2], idx_all, z128)
    hs = _tc_combine(parts, dinvt, W1, last=False)
    parts = _sc_gather_scatter(hs[0], hs[1], hs[2], idx_all, z128)
    hs = _tc_combine(parts, dinvt, W2, last=False)
    parts = _sc_gather_scatter(hs[0], hs[1], hs[2], idx_all, z128)
    out = _tc_combine(parts, dinvt, jnp.zeros((R, D, D), _f32), last=True)
    return out[0][:N]
